# Initial kernel scaffold; baseline (speedup 1.0000x reference)
import jax
import jax.numpy as jnp
from jax.experimental import pallas as pl


def kernel(img_embs, cap_embs, ln_g, ln_b, w1, b1, w2, b2, scale):
    raise NotImplementedError("write your pallas kernel here")


# speedup vs baseline: 3.1609x; 3.1609x over previous
import functools
import math

import jax
import jax.numpy as jnp
from jax.experimental import pallas as pl

_SPARSE_RATIO = 0.6
_INT_MIN = -(2 ** 31)


def _nt(a, b):
    return jax.lax.dot_general(a, b, (((1,), (1,)), ((), ())),
                               preferred_element_type=jnp.float32,
                               precision=jax.lax.Precision.HIGHEST)


def _body(img_ref, cap_ref, g_ref, bb_ref, w1_ref, b1_ref, w2_ref, b2_ref,
          s_ref, out_ref, *, bs, num_keep):
    L = img_ref.shape[1]
    C = img_ref.shape[2]
    K = w2_ref.shape[0]

    cap = cap_ref[0]
    capm = jnp.mean(cap, axis=0, keepdims=True)
    capn = jnp.sqrt(jnp.sum(capm * capm, axis=1, keepdims=True))
    cap_glo = capm / jnp.maximum(capn, 1e-12)

    ones_row = jnp.ones((1, C), jnp.float32)

    rows = []
    for b in range(bs):
        x = img_ref[b]
        xm = jnp.mean(x, axis=0, keepdims=True)
        xmn = jnp.sqrt(jnp.sum(xm * xm, axis=1, keepdims=True))
        glo = xm / jnp.maximum(xmn, 1e-12)
        q = glo + cap_glo
        dq = _nt(q, x)
        n2 = _nt(ones_row, x * x)
        rows.append(dq / jnp.maximum(jnp.sqrt(n2), 1e-12))
    S = jnp.concatenate(rows, axis=0)

    bi = jax.lax.bitcast_convert_type(S, jnp.int32)
    skey = bi ^ (jax.lax.shift_right_arithmetic(bi, 31) & jnp.int32(0x7FFFFFFF))
    ukey = skey ^ jnp.int32(_INT_MIN)
    prefix = jnp.zeros((bs, 1), jnp.int32)
    needed = jnp.full((bs, 1), num_keep, jnp.int32)
    for bit in range(31, -1, -1):
        bitv = jnp.int32(_INT_MIN if bit == 31 else (1 << bit))
        maskv = jnp.int32(-(1 << bit))
        cand = prefix | bitv
        hit = (ukey & maskv) == cand
        cnt = jnp.sum(hit.astype(jnp.int32), axis=1, keepdims=True)
        take = cnt >= needed
        prefix = jnp.where(take, cand, prefix)
        needed = jnp.where(take, needed, needed - cnt)
    thr = prefix ^ jnp.int32(_INT_MIN)
    gt = skey > thr
    eq = skey == thr
    io = jax.lax.broadcasted_iota(jnp.int32, (L, L), 0)
    jo = jax.lax.broadcasted_iota(jnp.int32, (L, L), 1)
    mle = (io <= jo).astype(jnp.float32)
    csum = jnp.dot(eq.astype(jnp.float32), mle,
                   preferred_element_type=jnp.float32,
                   precision=jax.lax.Precision.HIGHEST)
    keep = gt | (eq & (csum <= needed.astype(jnp.float32)))

    g = g_ref[...]
    bcol = bb_ref[...]
    w1 = w1_ref[...]
    b1 = b1_ref[...]
    w2 = w2_ref[...]
    b2 = b2_ref[...]
    scale = s_ref[0, 0]

    xf = img_ref[...].reshape(bs * L, C)
    mu = jnp.mean(xf, axis=1, keepdims=True)
    xc = xf - mu
    var = jnp.mean(xc * xc, axis=1, keepdims=True)
    xn = xc * jax.lax.rsqrt(var + 1e-5) * g + bcol
    pre = _nt(xn, w1) + b1
    h = 0.5 * pre * (1.0 + jax.lax.erf(pre * (1.0 / math.sqrt(2.0))))

    neg = jnp.float32(-jnp.inf)
    for b in range(bs):
        x = img_ref[b]
        hb = h[b * L:(b + 1) * L]
        logits_t = (_nt(w2, hb) + b2) * scale
        krow = keep[b:b + 1, :]
        lt = jnp.where(krow, logits_t, neg)
        mx = jnp.max(lt, axis=1, keepdims=True)
        e = jnp.exp(lt - mx)
        w = e / jnp.sum(e, axis=1, keepdims=True)
        srow = jnp.where(krow, neg, S[b:b + 1, :])
        m2 = jnp.max(srow, axis=1, keepdims=True)
        e2 = jnp.exp(srow - m2)
        wex = e2 / jnp.sum(e2, axis=1, keepdims=True)
        wall = jnp.concatenate([w, wex], axis=0)
        out_ref[b] = jnp.dot(wall, x, preferred_element_type=jnp.float32,
                             precision=jax.lax.Precision.HIGHEST)


def kernel(img_embs, cap_embs, ln_g, ln_b, w1, b1, w2, b2, scale):
    B_v, L_v, C = img_embs.shape
    B_t, L_t, _ = cap_embs.shape
    H = w1.shape[0]
    K = w2.shape[0]
    num_keep = math.ceil(L_v * _SPARSE_RATIO)
    bs = 8
    grid = (B_v // bs,)

    body = functools.partial(_body, bs=bs, num_keep=num_keep)
    out = pl.pallas_call(
        body,
        grid=grid,
        in_specs=[
            pl.BlockSpec((bs, L_v, C), lambda i: (i, 0, 0)),
            pl.BlockSpec((1, L_t, C), lambda i: (B_t - 1, 0, 0)),
            pl.BlockSpec((1, C), lambda i: (0, 0)),
            pl.BlockSpec((1, C), lambda i: (0, 0)),
            pl.BlockSpec((H, C), lambda i: (0, 0)),
            pl.BlockSpec((1, H), lambda i: (0, 0)),
            pl.BlockSpec((K, H), lambda i: (0, 0)),
            pl.BlockSpec((K, 1), lambda i: (0, 0)),
            pl.BlockSpec((1, 1), lambda i: (0, 0)),
        ],
        out_specs=pl.BlockSpec((bs, K + 1, C), lambda i: (i, 0, 0)),
        out_shape=jax.ShapeDtypeStruct((B_v, K + 1, C), jnp.float32),
    )(img_embs, cap_embs,
      ln_g.reshape(1, C), ln_b.reshape(1, C),
      w1, b1.reshape(1, H), w2, b2.reshape(K, 1),
      scale.reshape(1, 1))
    return out


# speedup vs baseline: 5.1975x; 1.6443x over previous
import functools
import math

import jax
import jax.numpy as jnp
from jax.experimental import pallas as pl

_SPARSE_RATIO = 0.6
_INT_MIN = -(2 ** 31)


def _nt(a, b, prec=jax.lax.Precision.DEFAULT):
    return jax.lax.dot_general(a, b, (((1,), (1,)), ((), ())),
                               preferred_element_type=jnp.float32,
                               precision=prec)


def _body(img_ref, cap_ref, g_ref, bb_ref, w1_ref, b1_ref, w2_ref, b2_ref,
          s_ref, out_ref, *, bs, num_keep):
    L = img_ref.shape[1]
    C = img_ref.shape[2]
    K = w2_ref.shape[0]

    cap = cap_ref[0]
    capm = jnp.mean(cap, axis=0, keepdims=True)
    capn = jnp.sqrt(jnp.sum(capm * capm, axis=1, keepdims=True))
    cap_glo = capm / jnp.maximum(capn, 1e-12)

    ones_row = jnp.ones((1, C), jnp.float32)

    rows = []
    for b in range(bs):
        x = img_ref[b]
        xm = jnp.mean(x, axis=0, keepdims=True)
        xmn = jnp.sqrt(jnp.sum(xm * xm, axis=1, keepdims=True))
        glo = xm / jnp.maximum(xmn, 1e-12)
        q = glo + cap_glo
        dq = _nt(q, x, jax.lax.Precision.HIGHEST)
        n2 = _nt(ones_row, x * x, jax.lax.Precision.HIGHEST)
        rows.append(dq / jnp.maximum(jnp.sqrt(n2), 1e-12))
    S = jnp.concatenate(rows, axis=0)

    bi = jax.lax.bitcast_convert_type(S, jnp.int32)
    skey = bi ^ (jax.lax.shift_right_arithmetic(bi, 31) & jnp.int32(0x7FFFFFFF))
    ukey = skey ^ jnp.int32(_INT_MIN)
    prefix = jnp.zeros((bs, 1), jnp.int32)
    needed = jnp.full((bs, 1), num_keep, jnp.int32)
    for bit in range(31, -1, -1):
        bitv = jnp.int32(_INT_MIN if bit == 31 else (1 << bit))
        maskv = jnp.int32(-(1 << bit))
        cand = prefix | bitv
        hit = (ukey & maskv) == cand
        cnt = jnp.sum(hit.astype(jnp.int32), axis=1, keepdims=True)
        take = cnt >= needed
        prefix = jnp.where(take, cand, prefix)
        needed = jnp.where(take, needed, needed - cnt)
    thr = prefix ^ jnp.int32(_INT_MIN)
    gt = skey > thr
    eq = skey == thr
    io = jax.lax.broadcasted_iota(jnp.int32, (L, L), 0)
    jo = jax.lax.broadcasted_iota(jnp.int32, (L, L), 1)
    mle = (io <= jo).astype(jnp.float32)
    csum = jnp.dot(eq.astype(jnp.float32), mle,
                   preferred_element_type=jnp.float32)
    keep = gt | (eq & (csum <= needed.astype(jnp.float32)))

    g = g_ref[...]
    bcol = bb_ref[...]
    w1 = w1_ref[...]
    b1 = b1_ref[...]
    w2 = w2_ref[...]
    b2 = b2_ref[...]
    scale = s_ref[0, 0]

    xf = img_ref[...].reshape(bs * L, C)
    mu = jnp.mean(xf, axis=1, keepdims=True)
    xc = xf - mu
    var = jnp.mean(xc * xc, axis=1, keepdims=True)
    xn = xc * jax.lax.rsqrt(var + 1e-5) * g + bcol
    pre = _nt(xn, w1) + b1
    h = 0.5 * pre * (1.0 + jax.lax.erf(pre * (1.0 / math.sqrt(2.0))))

    neg = jnp.float32(-jnp.inf)
    for b in range(bs):
        x = img_ref[b]
        hb = h[b * L:(b + 1) * L]
        logits_t = (_nt(w2, hb) + b2) * scale
        krow = keep[b:b + 1, :]
        lt = jnp.where(krow, logits_t, neg)
        mx = jnp.max(lt, axis=1, keepdims=True)
        e = jnp.exp(lt - mx)
        w = e / jnp.sum(e, axis=1, keepdims=True)
        srow = jnp.where(krow, neg, S[b:b + 1, :])
        m2 = jnp.max(srow, axis=1, keepdims=True)
        e2 = jnp.exp(srow - m2)
        wex = e2 / jnp.sum(e2, axis=1, keepdims=True)
        wall = jnp.concatenate([w, wex], axis=0)
        out_ref[b] = jnp.dot(wall, x, preferred_element_type=jnp.float32)


def kernel(img_embs, cap_embs, ln_g, ln_b, w1, b1, w2, b2, scale):
    B_v, L_v, C = img_embs.shape
    B_t, L_t, _ = cap_embs.shape
    H = w1.shape[0]
    K = w2.shape[0]
    num_keep = math.ceil(L_v * _SPARSE_RATIO)
    bs = 8
    grid = (B_v // bs,)

    body = functools.partial(_body, bs=bs, num_keep=num_keep)
    out = pl.pallas_call(
        body,
        grid=grid,
        in_specs=[
            pl.BlockSpec((bs, L_v, C), lambda i: (i, 0, 0)),
            pl.BlockSpec((1, L_t, C), lambda i: (B_t - 1, 0, 0)),
            pl.BlockSpec((1, C), lambda i: (0, 0)),
            pl.BlockSpec((1, C), lambda i: (0, 0)),
            pl.BlockSpec((H, C), lambda i: (0, 0)),
            pl.BlockSpec((1, H), lambda i: (0, 0)),
            pl.BlockSpec((K, H), lambda i: (0, 0)),
            pl.BlockSpec((K, 1), lambda i: (0, 0)),
            pl.BlockSpec((1, 1), lambda i: (0, 0)),
        ],
        out_specs=pl.BlockSpec((bs, K + 1, C), lambda i: (i, 0, 0)),
        out_shape=jax.ShapeDtypeStruct((B_v, K + 1, C), jnp.float32),
    )(img_embs, cap_embs,
      ln_g.reshape(1, C), ln_b.reshape(1, C),
      w1, b1.reshape(1, H), w2, b2.reshape(K, 1),
      scale.reshape(1, 1))
    return out


# speedup vs baseline: 7.8507x; 1.5105x over previous
import functools
import math

import jax
import jax.numpy as jnp
from jax.experimental import pallas as pl

_SPARSE_RATIO = 0.6
_INT_MIN = -(2 ** 31)
_HI = jax.lax.Precision.HIGHEST


def _nt(a, b, prec=jax.lax.Precision.DEFAULT):
    return jax.lax.dot_general(a, b, (((1,), (1,)), ((), ())),
                               preferred_element_type=jnp.float32,
                               precision=prec)


def _body(img_ref, cap_ref, g_ref, bb_ref, w1_ref, b1_ref, w2_ref, b2_ref,
          s_ref, out_ref, *, bs, L, num_keep):
    C = img_ref.shape[1]
    K = w2_ref.shape[0]

    cap = cap_ref[0]
    capm = jnp.mean(cap, axis=0, keepdims=True)
    capn = jnp.sqrt(jnp.sum(capm * capm, axis=1, keepdims=True))
    cap_glo = capm / jnp.maximum(capn, 1e-12)

    xf = img_ref[...]
    sq = xf * xf
    n2col = jnp.sum(sq, axis=1, keepdims=True)

    qs = []
    for b in range(bs):
        x = xf[b * L:(b + 1) * L]
        xm = jnp.mean(x, axis=0, keepdims=True)
        xmn = jnp.sqrt(jnp.sum(xm * xm, axis=1, keepdims=True))
        qs.append(xm / jnp.maximum(xmn, 1e-12) + cap_glo)
    qmat = jnp.concatenate(qs, axis=0)
    p = _nt(qmat, xf, _HI)
    dq = jnp.concatenate(
        [p[b:b + 1, b * L:(b + 1) * L] for b in range(bs)], axis=0)
    n2 = jnp.concatenate(
        [jnp.transpose(n2col[b * L:(b + 1) * L]) for b in range(bs)], axis=0)
    S = dq / jnp.maximum(jnp.sqrt(n2), 1e-12)

    bi = jax.lax.bitcast_convert_type(S, jnp.int32)
    skey = bi ^ (jax.lax.shift_right_arithmetic(bi, 31) & jnp.int32(0x7FFFFFFF))
    ukey = skey ^ jnp.int32(_INT_MIN)
    prefix = jnp.zeros((bs, 1), jnp.int32)
    needed = jnp.full((bs, 1), num_keep, jnp.int32)
    for bit in range(31, -1, -1):
        bitv = jnp.int32(_INT_MIN if bit == 31 else (1 << bit))
        maskv = jnp.int32(-(1 << bit))
        cand = prefix | bitv
        hit = (ukey & maskv) == cand
        cnt = jnp.sum(hit.astype(jnp.int32), axis=1, keepdims=True)
        take = cnt >= needed
        prefix = jnp.where(take, cand, prefix)
        needed = jnp.where(take, needed, needed - cnt)
    thr = prefix ^ jnp.int32(_INT_MIN)
    gt = skey > thr
    eq = skey == thr
    io = jax.lax.broadcasted_iota(jnp.int32, (L, L), 0)
    jo = jax.lax.broadcasted_iota(jnp.int32, (L, L), 1)
    mle = (io <= jo).astype(jnp.float32)
    csum = jnp.dot(eq.astype(jnp.float32), mle,
                   preferred_element_type=jnp.float32)
    keep = gt | (eq & (csum <= needed.astype(jnp.float32)))
    keepf = keep.astype(jnp.float32)

    g = g_ref[...]
    lb = bb_ref[...]
    w1 = w1_ref[...]
    b1 = b1_ref[...]
    w2 = w2_ref[...]
    b2 = b2_ref[...]
    scale = s_ref[0, 0]

    ones_col = jnp.ones((C, 1), jnp.float32)
    mu = jnp.dot(xf, ones_col) * (1.0 / C)
    rstd = jax.lax.rsqrt(n2col * (1.0 / C) - mu * mu + 1e-5)
    w1g = w1 * g
    w1g_rs = _nt(ones_row := jnp.ones((1, C), jnp.float32), w1g)
    bw1 = _nt(lb, w1)
    base = _nt(xf, w1g)
    pre = rstd * (base - mu * w1g_rs) + (bw1 + b1)
    h = 0.5 * pre * (1.0 + jax.lax.erf(pre * (1.0 / math.sqrt(2.0))))

    ones_l = jnp.ones((L, 1), jnp.float32)
    for b in range(bs):
        x = xf[b * L:(b + 1) * L]
        hb = h[b * L:(b + 1) * L]
        logits_t = (_nt(w2, hb) + b2) * scale
        krow = keepf[b:b + 1, :]
        mxs = jnp.max(jnp.max(logits_t, axis=0, keepdims=True),
                      axis=1, keepdims=True)
        e = jnp.exp(logits_t - mxs) * krow
        srow = S[b:b + 1, :]
        m2 = jnp.max(srow, axis=1, keepdims=True)
        e2 = jnp.exp(srow - m2) * (1.0 - krow)
        eall = jnp.concatenate([e, e2], axis=0)
        z = jnp.dot(eall, ones_l, preferred_element_type=jnp.float32)
        num = jnp.dot(eall, x, preferred_element_type=jnp.float32)
        out_ref[b] = num * (1.0 / z)


def kernel(img_embs, cap_embs, ln_g, ln_b, w1, b1, w2, b2, scale):
    B_v, L_v, C = img_embs.shape
    B_t, L_t, _ = cap_embs.shape
    H = w1.shape[0]
    K = w2.shape[0]
    num_keep = math.ceil(L_v * _SPARSE_RATIO)
    bs = 8
    grid = (B_v // bs,)

    body = functools.partial(_body, bs=bs, L=L_v, num_keep=num_keep)
    out = pl.pallas_call(
        body,
        grid=grid,
        in_specs=[
            pl.BlockSpec((bs * L_v, C), lambda i: (i, 0)),
            pl.BlockSpec((1, L_t, C), lambda i: (B_t - 1, 0, 0)),
            pl.BlockSpec((1, C), lambda i: (0, 0)),
            pl.BlockSpec((1, C), lambda i: (0, 0)),
            pl.BlockSpec((H, C), lambda i: (0, 0)),
            pl.BlockSpec((1, H), lambda i: (0, 0)),
            pl.BlockSpec((K, H), lambda i: (0, 0)),
            pl.BlockSpec((K, 1), lambda i: (0, 0)),
            pl.BlockSpec((1, 1), lambda i: (0, 0)),
        ],
        out_specs=pl.BlockSpec((bs, K + 1, C), lambda i: (i, 0, 0)),
        out_shape=jax.ShapeDtypeStruct((B_v, K + 1, C), jnp.float32),
    )(img_embs.reshape(B_v * L_v, C), cap_embs,
      ln_g.reshape(1, C), ln_b.reshape(1, C),
      w1, b1.reshape(1, H), w2, b2.reshape(K, 1),
      scale.reshape(1, 1))
    return out


# speedup vs baseline: 8.9636x; 1.1418x over previous
import functools
import math

import jax
import jax.numpy as jnp
from jax.experimental import pallas as pl

_SPARSE_RATIO = 0.6
_INT_MIN = -(2 ** 31)
_HI = jax.lax.Precision.HIGHEST


def _nt(a, b, prec=jax.lax.Precision.DEFAULT):
    return jax.lax.dot_general(a, b, (((1,), (1,)), ((), ())),
                               preferred_element_type=jnp.float32,
                               precision=prec)


def _body(img_ref, cap_ref, g_ref, bb_ref, w1_ref, b1_ref, w2_ref, b2_ref,
          s_ref, out_ref, *, bs, L, num_keep):
    C = img_ref.shape[1]
    K = w2_ref.shape[0]

    cap = cap_ref[0]
    capm = jnp.mean(cap, axis=0, keepdims=True)
    capn = jnp.sqrt(jnp.sum(capm * capm, axis=1, keepdims=True))
    cap_glo = capm / jnp.maximum(capn, 1e-12)

    xf = img_ref[...]
    sq = xf * xf
    n2col = jnp.sum(sq, axis=1, keepdims=True)

    qs = []
    for b in range(bs):
        x = xf[b * L:(b + 1) * L]
        xm = jnp.mean(x, axis=0, keepdims=True)
        xmn = jnp.sqrt(jnp.sum(xm * xm, axis=1, keepdims=True))
        qs.append(xm / jnp.maximum(xmn, 1e-12) + cap_glo)
    qmat = jnp.concatenate(qs, axis=0)
    dq_rows = []
    for gidx in range(bs // 8):
        pg = _nt(qmat[gidx * 8:(gidx + 1) * 8],
                 xf[gidx * 8 * L:(gidx + 1) * 8 * L], _HI)
        dq_rows.extend(pg[b:b + 1, b * L:(b + 1) * L] for b in range(8))
    dq = jnp.concatenate(dq_rows, axis=0)
    n2 = jnp.concatenate(
        [jnp.transpose(n2col[b * L:(b + 1) * L]) for b in range(bs)], axis=0)
    S = dq / jnp.maximum(jnp.sqrt(n2), 1e-12)

    bi = jax.lax.bitcast_convert_type(S, jnp.int32)
    skey = bi ^ (jax.lax.shift_right_arithmetic(bi, 31) & jnp.int32(0x7FFFFFFF))
    ukey = skey ^ jnp.int32(_INT_MIN)
    prefix = jnp.zeros((bs, 1), jnp.int32)
    needed = jnp.full((bs, 1), num_keep, jnp.int32)
    for bit in range(31, -1, -1):
        bitv = jnp.int32(_INT_MIN if bit == 31 else (1 << bit))
        maskv = jnp.int32(-(1 << bit))
        cand = prefix | bitv
        hit = (ukey & maskv) == cand
        cnt = jnp.sum(hit.astype(jnp.int32), axis=1, keepdims=True)
        take = cnt >= needed
        prefix = jnp.where(take, cand, prefix)
        needed = jnp.where(take, needed, needed - cnt)
    thr = prefix ^ jnp.int32(_INT_MIN)
    gt = skey > thr
    eq = skey == thr
    io = jax.lax.broadcasted_iota(jnp.int32, (L, L), 0)
    jo = jax.lax.broadcasted_iota(jnp.int32, (L, L), 1)
    mle = (io <= jo).astype(jnp.float32)
    csum = jnp.dot(eq.astype(jnp.float32), mle,
                   preferred_element_type=jnp.float32)
    keep = gt | (eq & (csum <= needed.astype(jnp.float32)))
    keepf = keep.astype(jnp.float32)

    g = g_ref[...]
    lb = bb_ref[...]
    w1 = w1_ref[...]
    b1 = b1_ref[...]
    w2 = w2_ref[...]
    b2 = b2_ref[...]
    scale = s_ref[0, 0]

    ones_col = jnp.ones((C, 1), jnp.float32)
    mu = jnp.dot(xf, ones_col) * (1.0 / C)
    rstd = jax.lax.rsqrt(n2col * (1.0 / C) - mu * mu + 1e-5)
    w1g = w1 * g
    w1g_rs = _nt(ones_row := jnp.ones((1, C), jnp.float32), w1g)
    bw1 = _nt(lb, w1)
    base = _nt(xf, w1g)
    pre = rstd * (base - mu * w1g_rs) + (bw1 + b1)
    h = 0.5 * pre * (1.0 + jax.lax.erf(pre * (1.0 / math.sqrt(2.0))))

    ones_l = jnp.ones((L, 1), jnp.float32)
    for b in range(bs):
        x = xf[b * L:(b + 1) * L]
        hb = h[b * L:(b + 1) * L]
        logits_t = (_nt(w2, hb) + b2) * scale
        krow = keepf[b:b + 1, :]
        mxs = jnp.max(jnp.max(logits_t, axis=0, keepdims=True),
                      axis=1, keepdims=True)
        e = jnp.exp(logits_t - mxs) * krow
        srow = S[b:b + 1, :]
        m2 = jnp.max(srow, axis=1, keepdims=True)
        e2 = jnp.exp(srow - m2) * (1.0 - krow)
        eall = jnp.concatenate([e, e2], axis=0)
        z = jnp.dot(eall, ones_l, preferred_element_type=jnp.float32)
        num = jnp.dot(eall, x, preferred_element_type=jnp.float32)
        out_ref[b] = num * (1.0 / z)


def kernel(img_embs, cap_embs, ln_g, ln_b, w1, b1, w2, b2, scale):
    B_v, L_v, C = img_embs.shape
    B_t, L_t, _ = cap_embs.shape
    H = w1.shape[0]
    K = w2.shape[0]
    num_keep = math.ceil(L_v * _SPARSE_RATIO)
    bs = 16
    grid = (B_v // bs,)

    body = functools.partial(_body, bs=bs, L=L_v, num_keep=num_keep)
    out = pl.pallas_call(
        body,
        grid=grid,
        in_specs=[
            pl.BlockSpec((bs * L_v, C), lambda i: (i, 0)),
            pl.BlockSpec((1, L_t, C), lambda i: (B_t - 1, 0, 0)),
            pl.BlockSpec((1, C), lambda i: (0, 0)),
            pl.BlockSpec((1, C), lambda i: (0, 0)),
            pl.BlockSpec((H, C), lambda i: (0, 0)),
            pl.BlockSpec((1, H), lambda i: (0, 0)),
            pl.BlockSpec((K, H), lambda i: (0, 0)),
            pl.BlockSpec((K, 1), lambda i: (0, 0)),
            pl.BlockSpec((1, 1), lambda i: (0, 0)),
        ],
        out_specs=pl.BlockSpec((bs, K + 1, C), lambda i: (i, 0, 0)),
        out_shape=jax.ShapeDtypeStruct((B_v, K + 1, C), jnp.float32),
    )(img_embs.reshape(B_v * L_v, C), cap_embs,
      ln_g.reshape(1, C), ln_b.reshape(1, C),
      w1, b1.reshape(1, H), w2, b2.reshape(K, 1),
      scale.reshape(1, 1))
    return out


# speedup vs baseline: 12.3064x; 1.3729x over previous
import functools
import math

import jax
import jax.numpy as jnp
from jax.experimental import pallas as pl

_SPARSE_RATIO = 0.6
_INT_MIN = -(2 ** 31)
_HI = jax.lax.Precision.HIGHEST


def _nt(a, b, prec=jax.lax.Precision.DEFAULT):
    return jax.lax.dot_general(a, b, (((1,), (1,)), ((), ())),
                               preferred_element_type=jnp.float32,
                               precision=prec)


def _body(img_ref, cap_ref, g_ref, bb_ref, w1_ref, b1_ref, w2_ref, b2_ref,
          s_ref, out_ref, *, bs, L, num_keep):
    C = img_ref.shape[1]
    K = w2_ref.shape[0]

    cap = cap_ref[0]
    capm = jnp.mean(cap, axis=0, keepdims=True)
    capn = jnp.sqrt(jnp.sum(capm * capm, axis=1, keepdims=True))
    cap_glo = capm / jnp.maximum(capn, 1e-12)

    xf = img_ref[...]
    sq = xf * xf
    n2col = jnp.sum(sq, axis=1, keepdims=True)

    dq_rows, n2_rows = [], []
    for b in range(bs):
        x = xf[b * L:(b + 1) * L]
        xm = jnp.mean(x, axis=0, keepdims=True)
        xmn = jnp.sqrt(jnp.sum(xm * xm, axis=1, keepdims=True))
        q = xm / jnp.maximum(xmn, 1e-12) + cap_glo
        dqcol = jnp.sum(x * q, axis=1, keepdims=True)
        two = jnp.concatenate([dqcol, n2col[b * L:(b + 1) * L]], axis=1)
        t = jnp.transpose(two)
        dq_rows.append(t[0:1])
        n2_rows.append(t[1:2])
    dq = jnp.concatenate(dq_rows, axis=0)
    n2 = jnp.concatenate(n2_rows, axis=0)
    S = dq / jnp.maximum(jnp.sqrt(n2), 1e-12)

    bi = jax.lax.bitcast_convert_type(S, jnp.int32)
    skey = bi ^ (jax.lax.shift_right_arithmetic(bi, 31) & jnp.int32(0x7FFFFFFF))
    ukey = skey ^ jnp.int32(_INT_MIN)
    prefix = jnp.zeros((bs, 1), jnp.int32)
    needed = jnp.full((bs, 1), num_keep, jnp.int32)
    for bit in range(31, -1, -1):
        bitv = jnp.int32(_INT_MIN if bit == 31 else (1 << bit))
        maskv = jnp.int32(-(1 << bit))
        cand = prefix | bitv
        hit = (ukey & maskv) == cand
        cnt = jnp.sum(hit.astype(jnp.int32), axis=1, keepdims=True)
        take = cnt >= needed
        prefix = jnp.where(take, cand, prefix)
        needed = jnp.where(take, needed, needed - cnt)
    thr = prefix ^ jnp.int32(_INT_MIN)
    gt = skey > thr
    eq = skey == thr
    io = jax.lax.broadcasted_iota(jnp.int32, (L, L), 0)
    jo = jax.lax.broadcasted_iota(jnp.int32, (L, L), 1)
    mle = (io <= jo).astype(jnp.float32)
    csum = jnp.dot(eq.astype(jnp.float32), mle,
                   preferred_element_type=jnp.float32)
    keep = gt | (eq & (csum <= needed.astype(jnp.float32)))
    keepf = keep.astype(jnp.float32)

    g = g_ref[...]
    lb = bb_ref[...]
    w1 = w1_ref[...]
    b1 = b1_ref[...]
    w2 = w2_ref[...]
    b2 = b2_ref[...]
    scale = s_ref[0, 0]

    ones_col = jnp.ones((C, 1), jnp.float32)
    mu = jnp.dot(xf, ones_col) * (1.0 / C)
    rstd = jax.lax.rsqrt(n2col * (1.0 / C) - mu * mu + 1e-5)
    w1g = w1 * g
    w1g_rs = _nt(ones_row := jnp.ones((1, C), jnp.float32), w1g)
    bw1 = _nt(lb, w1)
    base = _nt(xf, w1g)
    pre = rstd * (base - mu * w1g_rs) + (bw1 + b1)
    h = 0.5 * pre * (1.0 + jax.lax.erf(pre * (1.0 / math.sqrt(2.0))))

    ones_l = jnp.ones((L, 1), jnp.float32)
    for b in range(bs):
        x = xf[b * L:(b + 1) * L]
        hb = h[b * L:(b + 1) * L]
        logits_t = (_nt(w2, hb) + b2) * scale
        krow = keepf[b:b + 1, :]
        mxs = jnp.max(jnp.max(logits_t, axis=0, keepdims=True),
                      axis=1, keepdims=True)
        e = jnp.exp(logits_t - mxs) * krow
        srow = S[b:b + 1, :]
        m2 = jnp.max(srow, axis=1, keepdims=True)
        e2 = jnp.exp(srow - m2) * (1.0 - krow)
        eall = jnp.concatenate([e, e2], axis=0)
        z = jnp.dot(eall, ones_l, preferred_element_type=jnp.float32)
        num = jnp.dot(eall, x, preferred_element_type=jnp.float32)
        out_ref[b] = num * (1.0 / z)


def kernel(img_embs, cap_embs, ln_g, ln_b, w1, b1, w2, b2, scale):
    B_v, L_v, C = img_embs.shape
    B_t, L_t, _ = cap_embs.shape
    H = w1.shape[0]
    K = w2.shape[0]
    num_keep = math.ceil(L_v * _SPARSE_RATIO)
    bs = 16
    grid = (B_v // bs,)

    body = functools.partial(_body, bs=bs, L=L_v, num_keep=num_keep)
    out = pl.pallas_call(
        body,
        grid=grid,
        in_specs=[
            pl.BlockSpec((bs * L_v, C), lambda i: (i, 0)),
            pl.BlockSpec((1, L_t, C), lambda i: (B_t - 1, 0, 0)),
            pl.BlockSpec((1, C), lambda i: (0, 0)),
            pl.BlockSpec((1, C), lambda i: (0, 0)),
            pl.BlockSpec((H, C), lambda i: (0, 0)),
            pl.BlockSpec((1, H), lambda i: (0, 0)),
            pl.BlockSpec((K, H), lambda i: (0, 0)),
            pl.BlockSpec((K, 1), lambda i: (0, 0)),
            pl.BlockSpec((1, 1), lambda i: (0, 0)),
        ],
        out_specs=pl.BlockSpec((bs, K + 1, C), lambda i: (i, 0, 0)),
        out_shape=jax.ShapeDtypeStruct((B_v, K + 1, C), jnp.float32),
    )(img_embs.reshape(B_v * L_v, C), cap_embs,
      ln_g.reshape(1, C), ln_b.reshape(1, C),
      w1, b1.reshape(1, H), w2, b2.reshape(K, 1),
      scale.reshape(1, 1))
    return out


# speedup vs baseline: 12.3619x; 1.0045x over previous
import functools
import math

import jax
import jax.numpy as jnp
from jax.experimental import pallas as pl

_SPARSE_RATIO = 0.6
_INT_MIN = -(2 ** 31)
_HI = jax.lax.Precision.HIGHEST


def _nt(a, b, prec=jax.lax.Precision.DEFAULT):
    return jax.lax.dot_general(a, b, (((1,), (1,)), ((), ())),
                               preferred_element_type=jnp.float32,
                               precision=prec)


def _body(img_ref, cap_ref, g_ref, bb_ref, w1_ref, b1_ref, w2_ref, b2_ref,
          s_ref, out_ref, *, bs, L, num_keep):
    C = img_ref.shape[1]
    K = w2_ref.shape[0]

    cap = cap_ref[0]
    capm = jnp.mean(cap, axis=0, keepdims=True)
    capn = jnp.sqrt(jnp.sum(capm * capm, axis=1, keepdims=True))
    cap_glo = capm / jnp.maximum(capn, 1e-12)

    xf = img_ref[...]
    xb = xf.astype(jnp.bfloat16)
    sq = xf * xf
    n2col = jnp.sum(sq, axis=1, keepdims=True)

    dq_rows, n2_rows = [], []
    for b in range(bs):
        x = xf[b * L:(b + 1) * L]
        xm = jnp.mean(x, axis=0, keepdims=True)
        xmn = jnp.sqrt(jnp.sum(xm * xm, axis=1, keepdims=True))
        q = xm / jnp.maximum(xmn, 1e-12) + cap_glo
        dqcol = jnp.sum(x * q, axis=1, keepdims=True)
        two = jnp.concatenate([dqcol, n2col[b * L:(b + 1) * L]], axis=1)
        t = jnp.transpose(two)
        dq_rows.append(t[0:1])
        n2_rows.append(t[1:2])
    dq = jnp.concatenate(dq_rows, axis=0)
    n2 = jnp.concatenate(n2_rows, axis=0)
    S = dq / jnp.maximum(jnp.sqrt(n2), 1e-12)

    bi = jax.lax.bitcast_convert_type(S, jnp.int32)
    skey = bi ^ (jax.lax.shift_right_arithmetic(bi, 31) & jnp.int32(0x7FFFFFFF))
    ukey = skey ^ jnp.int32(_INT_MIN)
    prefix = jnp.zeros((bs, 1), jnp.int32)
    needed = jnp.full((bs, 1), num_keep, jnp.int32)
    for bit in range(31, -1, -1):
        bitv = jnp.int32(_INT_MIN if bit == 31 else (1 << bit))
        maskv = jnp.int32(-(1 << bit))
        cand = prefix | bitv
        hit = (ukey & maskv) == cand
        cnt = jnp.sum(hit.astype(jnp.int32), axis=1, keepdims=True)
        take = cnt >= needed
        prefix = jnp.where(take, cand, prefix)
        needed = jnp.where(take, needed, needed - cnt)
    thr = prefix ^ jnp.int32(_INT_MIN)
    gt = skey > thr
    eq = skey == thr
    io = jax.lax.broadcasted_iota(jnp.int32, (L, L), 0)
    jo = jax.lax.broadcasted_iota(jnp.int32, (L, L), 1)
    mle = (io <= jo).astype(jnp.float32)
    csum = jnp.dot(eq.astype(jnp.float32), mle,
                   preferred_element_type=jnp.float32)
    keep = gt | (eq & (csum <= needed.astype(jnp.float32)))
    keepf = keep.astype(jnp.float32)

    g = g_ref[...]
    lb = bb_ref[...]
    w1 = w1_ref[...]
    b1 = b1_ref[...]
    w2 = w2_ref[...]
    b2 = b2_ref[...]
    scale = s_ref[0, 0]

    ones_col = jnp.ones((C, 1), jnp.bfloat16)
    mu = jnp.dot(xb, ones_col,
                 preferred_element_type=jnp.float32) * (1.0 / C)
    rstd = jax.lax.rsqrt(n2col * (1.0 / C) - mu * mu + 1e-5)
    w1g = w1 * g
    w1g_rs = _nt(ones_row := jnp.ones((1, C), jnp.float32), w1g)
    bw1 = _nt(lb, w1)
    base = _nt(xb, w1g.astype(jnp.bfloat16))
    pre = rstd * (base - mu * w1g_rs) + (bw1 + b1)
    h = 0.5 * pre * (1.0 + jax.lax.erf(pre * (1.0 / math.sqrt(2.0))))

    ones_l = jnp.ones((L, 1), jnp.bfloat16)
    for b in range(bs):
        x16 = xb[b * L:(b + 1) * L]
        hb = h[b * L:(b + 1) * L]
        logits_t = (_nt(w2, hb) + b2) * scale
        krow = keepf[b:b + 1, :]
        mxs = jnp.max(jnp.max(logits_t, axis=0, keepdims=True),
                      axis=1, keepdims=True)
        e = jnp.exp(logits_t - mxs) * krow
        srow = S[b:b + 1, :]
        m2 = jnp.max(srow, axis=1, keepdims=True)
        e2 = jnp.exp(srow - m2) * (1.0 - krow)
        eall = jnp.concatenate([e, e2], axis=0).astype(jnp.bfloat16)
        z = jnp.dot(eall, ones_l, preferred_element_type=jnp.float32)
        num = jnp.dot(eall, x16, preferred_element_type=jnp.float32)
        out_ref[b] = num * (1.0 / z)


def kernel(img_embs, cap_embs, ln_g, ln_b, w1, b1, w2, b2, scale):
    B_v, L_v, C = img_embs.shape
    B_t, L_t, _ = cap_embs.shape
    H = w1.shape[0]
    K = w2.shape[0]
    num_keep = math.ceil(L_v * _SPARSE_RATIO)
    bs = 16
    grid = (B_v // bs,)

    body = functools.partial(_body, bs=bs, L=L_v, num_keep=num_keep)
    out = pl.pallas_call(
        body,
        grid=grid,
        in_specs=[
            pl.BlockSpec((bs * L_v, C), lambda i: (i, 0)),
            pl.BlockSpec((1, L_t, C), lambda i: (B_t - 1, 0, 0)),
            pl.BlockSpec((1, C), lambda i: (0, 0)),
            pl.BlockSpec((1, C), lambda i: (0, 0)),
            pl.BlockSpec((H, C), lambda i: (0, 0)),
            pl.BlockSpec((1, H), lambda i: (0, 0)),
            pl.BlockSpec((K, H), lambda i: (0, 0)),
            pl.BlockSpec((K, 1), lambda i: (0, 0)),
            pl.BlockSpec((1, 1), lambda i: (0, 0)),
        ],
        out_specs=pl.BlockSpec((bs, K + 1, C), lambda i: (i, 0, 0)),
        out_shape=jax.ShapeDtypeStruct((B_v, K + 1, C), jnp.float32),
    )(img_embs.reshape(B_v * L_v, C), cap_embs,
      ln_g.reshape(1, C), ln_b.reshape(1, C),
      w1, b1.reshape(1, H), w2, b2.reshape(K, 1),
      scale.reshape(1, 1))
    return out


# speedup vs baseline: 12.3964x; 1.0028x over previous
import functools
import math

import jax
import jax.numpy as jnp
from jax.experimental import pallas as pl

_SPARSE_RATIO = 0.6
_INT_MIN = -(2 ** 31)
_HI = jax.lax.Precision.HIGHEST


def _nt(a, b, prec=jax.lax.Precision.DEFAULT):
    return jax.lax.dot_general(a, b, (((1,), (1,)), ((), ())),
                               preferred_element_type=jnp.float32,
                               precision=prec)


def _body(img_ref, cap_ref, g_ref, bb_ref, w1_ref, b1_ref, w2_ref, b2_ref,
          s_ref, out_ref, *, bs, L, num_keep):
    C = img_ref.shape[1]
    K = w2_ref.shape[0]

    cap = cap_ref[0]
    capm = jnp.mean(cap, axis=0, keepdims=True)
    capn = jnp.sqrt(jnp.sum(capm * capm, axis=1, keepdims=True))
    cap_glo = capm / jnp.maximum(capn, 1e-12)

    xb = img_ref[...].astype(jnp.bfloat16)

    dq_rows, n2_rows = [], []
    n2cols = []
    for b in range(bs):
        x = img_ref[b * L:(b + 1) * L]
        n2c = jnp.sum(x * x, axis=1, keepdims=True)
        n2cols.append(n2c)
        xm = jnp.mean(x, axis=0, keepdims=True)
        xmn = jnp.sqrt(jnp.sum(xm * xm, axis=1, keepdims=True))
        q = xm / jnp.maximum(xmn, 1e-12) + cap_glo
        dqcol = jnp.sum(x * q, axis=1, keepdims=True)
        two = jnp.concatenate([dqcol, n2c], axis=1)
        t = jnp.transpose(two)
        dq_rows.append(t[0:1])
        n2_rows.append(t[1:2])
    n2col = jnp.concatenate(n2cols, axis=0)
    dq = jnp.concatenate(dq_rows, axis=0)
    n2 = jnp.concatenate(n2_rows, axis=0)
    S = dq / jnp.maximum(jnp.sqrt(n2), 1e-12)

    bi = jax.lax.bitcast_convert_type(S, jnp.int32)
    skey = bi ^ (jax.lax.shift_right_arithmetic(bi, 31) & jnp.int32(0x7FFFFFFF))
    ukey = skey ^ jnp.int32(_INT_MIN)
    prefix = jnp.zeros((bs, 1), jnp.int32)
    needed = jnp.full((bs, 1), num_keep, jnp.int32)
    for bit in range(31, -1, -1):
        bitv = jnp.int32(_INT_MIN if bit == 31 else (1 << bit))
        maskv = jnp.int32(-(1 << bit))
        cand = prefix | bitv
        hit = (ukey & maskv) == cand
        cnt = jnp.sum(hit.astype(jnp.int32), axis=1, keepdims=True)
        take = cnt >= needed
        prefix = jnp.where(take, cand, prefix)
        needed = jnp.where(take, needed, needed - cnt)
    thr = prefix ^ jnp.int32(_INT_MIN)
    gt = skey > thr
    eq = skey == thr
    io = jax.lax.broadcasted_iota(jnp.int32, (L, L), 0)
    jo = jax.lax.broadcasted_iota(jnp.int32, (L, L), 1)
    mle = (io <= jo).astype(jnp.float32)
    csum = jnp.dot(eq.astype(jnp.float32), mle,
                   preferred_element_type=jnp.float32)
    keep = gt | (eq & (csum <= needed.astype(jnp.float32)))
    keepf = keep.astype(jnp.float32)

    g = g_ref[...]
    lb = bb_ref[...]
    w1 = w1_ref[...]
    b1 = b1_ref[...]
    w2 = w2_ref[...]
    b2 = b2_ref[...]
    scale = s_ref[0, 0]

    ones_col = jnp.ones((C, 1), jnp.bfloat16)
    mu = jnp.dot(xb, ones_col,
                 preferred_element_type=jnp.float32) * (1.0 / C)
    rstd = jax.lax.rsqrt(n2col * (1.0 / C) - mu * mu + 1e-5)
    w1g = w1 * g
    w1g_rs = _nt(ones_row := jnp.ones((1, C), jnp.float32), w1g)
    bw1 = _nt(lb, w1)
    base = _nt(xb, w1g.astype(jnp.bfloat16))
    pre = rstd * (base - mu * w1g_rs) + (bw1 + b1)
    h = pre * (0.5 * jax.lax.erf(pre * (1.0 / math.sqrt(2.0))) + 0.5)

    ones_l = jnp.ones((L, 1), jnp.bfloat16)
    for b in range(bs):
        x16 = xb[b * L:(b + 1) * L]
        hb = h[b * L:(b + 1) * L]
        logits_t = (_nt(w2, hb) + b2) * scale
        krow = keepf[b:b + 1, :]
        mxs = jnp.max(jnp.max(logits_t, axis=0, keepdims=True),
                      axis=1, keepdims=True)
        e = jnp.exp(logits_t - mxs) * krow
        srow = S[b:b + 1, :]
        m2 = jnp.max(srow, axis=1, keepdims=True)
        e2 = jnp.exp(srow - m2) * (1.0 - krow)
        eall = jnp.concatenate([e, e2], axis=0).astype(jnp.bfloat16)
        z = jnp.dot(eall, ones_l, preferred_element_type=jnp.float32)
        num = jnp.dot(eall, x16, preferred_element_type=jnp.float32)
        out_ref[b] = num * (1.0 / z)


def kernel(img_embs, cap_embs, ln_g, ln_b, w1, b1, w2, b2, scale):
    B_v, L_v, C = img_embs.shape
    B_t, L_t, _ = cap_embs.shape
    H = w1.shape[0]
    K = w2.shape[0]
    num_keep = math.ceil(L_v * _SPARSE_RATIO)
    bs = 16
    grid = (B_v // bs,)

    body = functools.partial(_body, bs=bs, L=L_v, num_keep=num_keep)
    out = pl.pallas_call(
        body,
        grid=grid,
        in_specs=[
            pl.BlockSpec((bs * L_v, C), lambda i: (i, 0)),
            pl.BlockSpec((1, L_t, C), lambda i: (B_t - 1, 0, 0)),
            pl.BlockSpec((1, C), lambda i: (0, 0)),
            pl.BlockSpec((1, C), lambda i: (0, 0)),
            pl.BlockSpec((H, C), lambda i: (0, 0)),
            pl.BlockSpec((1, H), lambda i: (0, 0)),
            pl.BlockSpec((K, H), lambda i: (0, 0)),
            pl.BlockSpec((K, 1), lambda i: (0, 0)),
            pl.BlockSpec((1, 1), lambda i: (0, 0)),
        ],
        out_specs=pl.BlockSpec((bs, K + 1, C), lambda i: (i, 0, 0)),
        out_shape=jax.ShapeDtypeStruct((B_v, K + 1, C), jnp.float32),
    )(img_embs.reshape(B_v * L_v, C), cap_embs,
      ln_g.reshape(1, C), ln_b.reshape(1, C),
      w1, b1.reshape(1, H), w2, b2.reshape(K, 1),
      scale.reshape(1, 1))
    return out
